# Initial kernel scaffold; baseline (speedup 1.0000x reference)
import jax
import jax.numpy as jnp
from jax.experimental import pallas as pl


def kernel(x, edge_attr, cond, glob, W_msg, b_msg, W_upd, b_upd, edge_index):
    raise NotImplementedError("write your pallas kernel here")


# speedup vs baseline: 3.3587x; 3.3587x over previous
import functools

import jax
import jax.numpy as jnp
from jax import lax
from jax.experimental import pallas as pl
from jax.experimental.pallas import tpu as pltpu
from jax.experimental.pallas import tpu_sc as plsc

N = 10000
E = 320000
D = 128
DE = 4
NC = 1
NG = 8

SC_CORES = 2
SC_TILES = 16
NW = SC_CORES * SC_TILES
CHUNK = 128
NCHUNK = E // CHUNK
CHUNKS_PER_TILE = (NCHUNK + NW - 1) // NW
N_PAD = 10240
ROWS_PER_TILE = N_PAD // SC_TILES


def _xm_body(x_ref, w1_ref, b_ref, o_ref):
    o_ref[...] = (
        jnp.dot(x_ref[...], w1_ref[...], preferred_element_type=jnp.float32)
        + b_ref[...]
    )


def _em_body(ea_ref, w2_ref, o_ref):
    o_ref[...] = lax.dot_general(
        ea_ref[...], w2_ref[...], (((1,), (0,)), ((), ())),
        preferred_element_type=jnp.float32,
    )


def _sc_agg_body(xm_hbm, em_hbm, src_hbm, dst_hbm, out_hbm,
                 idx_s, idx_d, rows_v, em_v, agg_sh, gsem):
    cid = lax.axis_index("c")
    sid = lax.axis_index("s")
    wid = sid * SC_CORES + cid

    zvec = jnp.zeros((16,), jnp.float32)

    def zero_body(i, _):
        r = i // (D // 16)
        j = i % (D // 16)
        rows_v[r, pl.ds(j * 16, 16)] = zvec
        return 0

    lax.fori_loop(0, CHUNK * (D // 16), zero_body, 0)
    for i in range(ROWS_PER_TILE // CHUNK):
        pltpu.sync_copy(
            rows_v,
            agg_sh.at[pl.ds(sid * ROWS_PER_TILE + i * CHUNK, CHUNK)],
        )
    plsc.subcore_barrier()

    def chunk_body(i, _):
        c0 = wid + i * NW

        @pl.when(c0 < NCHUNK)
        def _():
            base = c0 * CHUNK
            pltpu.sync_copy(src_hbm.at[pl.ds(base, CHUNK)], idx_s)
            pltpu.sync_copy(dst_hbm.at[pl.ds(base, CHUNK)], idx_d)
            gather = pltpu.async_copy(xm_hbm.at[idx_s], rows_v, gsem)
            pltpu.sync_copy(em_hbm.at[pl.ds(base, CHUNK)], em_v)
            gather.wait()

            def row_body(r, _):
                for j in range(D // 16):
                    v = rows_v[r, pl.ds(j * 16, 16)] + em_v[r, pl.ds(j * 16, 16)]
                    rows_v[r, pl.ds(j * 16, 16)] = jnp.maximum(v, 0.0)
                return 0

            lax.fori_loop(0, CHUNK, row_body, 0)
            pltpu.sync_copy(rows_v, agg_sh.at[idx_d], add=True)

        return 0

    lax.fori_loop(0, CHUNKS_PER_TILE, chunk_body, 0)
    plsc.subcore_barrier()

    pltpu.sync_copy(
        agg_sh.at[pl.ds(sid * ROWS_PER_TILE, ROWS_PER_TILE)],
        out_hbm.at[cid, pl.ds(sid * ROWS_PER_TILE, ROWS_PER_TILE)],
    )


_sc_agg = functools.partial(
    pl.kernel,
    out_type=jax.ShapeDtypeStruct((SC_CORES, N_PAD, D), jnp.float32),
    mesh=plsc.VectorSubcoreMesh(
        core_axis_name="c", subcore_axis_name="s",
        num_cores=SC_CORES, num_subcores=SC_TILES,
    ),
    scratch_types=[
        pltpu.VMEM((CHUNK,), jnp.int32),
        pltpu.VMEM((CHUNK,), jnp.int32),
        pltpu.VMEM((CHUNK, D), jnp.float32),
        pltpu.VMEM((CHUNK, D), jnp.float32),
        pltpu.VMEM_SHARED((N_PAD, D), jnp.float32),
        pltpu.SemaphoreType.DMA,
    ],
)(_sc_agg_body)


def _upd_body(x_ref, a0_ref, a1_ref, cond_ref, glob_ref,
              wx_ref, wa_ref, wc_ref, wg_ref, b_ref, o_ref):
    acc = jnp.dot(x_ref[...], wx_ref[...], preferred_element_type=jnp.float32)
    agg = a0_ref[...] + a1_ref[...]
    acc += jnp.dot(agg, wa_ref[...], preferred_element_type=jnp.float32)
    acc += cond_ref[...] * wc_ref[...]
    acc += jnp.dot(glob_ref[...], wg_ref[...], preferred_element_type=jnp.float32)
    o_ref[...] = jnp.maximum(acc + b_ref[...], 0.0)


def kernel(x, edge_attr, cond, glob, W_msg, b_msg, W_upd, b_upd, edge_index):
    src = edge_index[0].astype(jnp.int32)
    dst = edge_index[1].astype(jnp.int32)
    w1 = W_msg[:D]
    w2 = W_msg[D:]
    b_msg2 = b_msg.reshape(1, D)
    wx = W_upd[:D]
    wa = W_upd[D:2 * D]
    wc = W_upd[2 * D:2 * D + NC]
    wg = W_upd[2 * D + NC:]
    b_upd2 = b_upd.reshape(1, D)

    xm = pl.pallas_call(
        _xm_body,
        out_shape=jax.ShapeDtypeStruct((N, D), jnp.float32),
    )(x, w1, b_msg2)

    em = pl.pallas_call(
        _em_body,
        grid=(E // 8000,),
        in_specs=[
            pl.BlockSpec((8000, DE), lambda i: (i, 0)),
            pl.BlockSpec((DE, D), lambda i: (0, 0)),
        ],
        out_specs=pl.BlockSpec((8000, D), lambda i: (i, 0)),
        out_shape=jax.ShapeDtypeStruct((E, D), jnp.float32),
    )(edge_attr, w2)

    agg2 = _sc_agg(xm, em, src, dst)

    out = pl.pallas_call(
        _upd_body,
        out_shape=jax.ShapeDtypeStruct((N, D), jnp.float32),
    )(x, agg2[0, :N], agg2[1, :N], cond, glob, wx, wa, wc, wg, b_upd2)
    return out
